# baseline (speedup 1.0000x reference)
import functools
import math

import jax
import jax.numpy as jnp
import numpy as np
from jax import lax
from jax.experimental import pallas as pl
from jax.experimental.pallas import tpu as pltpu
from jax.experimental.pallas import tpu_sc as plsc

N, NN, F, K, MAXZ = 10000, 32, 128, 32, 87
E = N * NN
ZP = 96
PD = 128

NC, NS = 2, 16
NW = NC * NS
EPW = E // NW

_LOGBINOM = np.array(
    [math.lgamma(K) - math.lgamma(i + 1.0) - math.lgamma(K - i) for i in range(K)],
    dtype=np.float32,
)


def _sw(x):
    return x * jax.nn.sigmoid(x)


def _res(x, w1, b1, w2, b2):
    y = jnp.dot(_sw(x), w1, preferred_element_type=jnp.float32) + b1
    y = jnp.dot(_sw(y), w2, preferred_element_type=jnp.float32) + b2
    return x + y


@functools.lru_cache(maxsize=None)
def _make_sc_gather(d, chunk, dtype_name, ne):
    dtype = jnp.dtype(dtype_name)
    mesh = plsc.VectorSubcoreMesh(core_axis_name="c", subcore_axis_name="s")
    epw = ne // NW
    nch = epw // chunk
    npairs = nch // 2
    odd = nch % 2 == 1
    assert epw % chunk == 0 and chunk % 8 == 0 and nch >= 2

    @functools.partial(
        pl.kernel,
        mesh=mesh,
        out_type=jax.ShapeDtypeStruct((ne, d), dtype),
        scratch_types=[
            pltpu.VMEM((epw,), jnp.int32),
            pltpu.VMEM((2, chunk, d), dtype),
            pltpu.SemaphoreType.DMA,
            pltpu.SemaphoreType.DMA,
            pltpu.SemaphoreType.DMA,
            pltpu.SemaphoreType.DMA,
        ],
    )
    def gather_k(idx_hbm, table_hbm, out_hbm, idx_v, rows_v, g0, g1, w0, w1):
        wid = lax.axis_index("s") * NC + lax.axis_index("c")
        base = wid * epw
        pltpu.sync_copy(idx_hbm.at[pl.ds(base, epw)], idx_v)

        def fire_g(c, slot, sem):
            pltpu.async_copy(
                table_hbm.at[idx_v.at[pl.ds(c * chunk, chunk)]],
                rows_v.at[slot], sem)

        def wait_g(slot, sem):
            pltpu.make_async_copy(
                table_hbm.at[idx_v.at[pl.ds(0, chunk)]], rows_v.at[slot], sem
            ).wait()

        def fire_w(c, slot, sem):
            pltpu.async_copy(
                rows_v.at[slot], out_hbm.at[pl.ds(base + c * chunk, chunk)],
                sem)

        def wait_w(slot, sem):
            pltpu.make_async_copy(
                rows_v.at[slot], out_hbm.at[pl.ds(base, chunk)], sem
            ).wait()

        fire_g(0, 0, g0)

        def pair(g, carry):
            c0 = 2 * g
            wait_g(0, g0)
            fire_w(c0, 0, w0)

            @pl.when(g > 0)
            def _():
                wait_w(1, w1)

            fire_g(c0 + 1, 1, g1)
            wait_g(1, g1)
            fire_w(c0 + 1, 1, w1)
            wait_w(0, w0)

            @pl.when(c0 + 2 < nch)
            def _():
                fire_g(c0 + 2, 0, g0)

            return carry

        lax.fori_loop(0, npairs, pair, 0)
        if odd:
            wait_g(0, g0)
            fire_w(nch - 1, 0, w0)
            wait_w(0, w0)
        if npairs > 0:
            wait_w(1, w1)

    return gather_k


def _make_sc_dist(chunk):
    mesh = plsc.VectorSubcoreMesh(core_axis_name="c", subcore_axis_name="s")
    nlane = 16

    @functools.partial(
        pl.kernel,
        mesh=mesh,
        out_type=jax.ShapeDtypeStruct((E,), jnp.float32),
        compiler_params=pltpu.CompilerParams(needs_layout_passes=False),
        scratch_types=[
            pltpu.VMEM((N,), jnp.float32),
            pltpu.VMEM((N,), jnp.float32),
            pltpu.VMEM((N,), jnp.float32),
            pltpu.VMEM((chunk,), jnp.int32),
            pltpu.VMEM((chunk,), jnp.float32),
        ],
    )
    def dist_k(idx_hbm, px_hbm, py_hbm, pz_hbm, out_hbm,
               px_v, py_v, pz_v, idx_v, d2_v):
        wid = lax.axis_index("s") * NC + lax.axis_index("c")
        base = wid * EPW
        pltpu.sync_copy(px_hbm, px_v)
        pltpu.sync_copy(py_hbm, py_v)
        pltpu.sync_copy(pz_hbm, pz_v)

        def chunk_step(i, carry):
            off = base + i * chunk
            pltpu.sync_copy(idx_hbm.at[pl.ds(off, chunk)], idx_v)

            def vec_step(j, c2):
                jv = lax.iota(jnp.int32, nlane) + (off + j * nlane)
                ai = lax.shift_right_logical(jv, 5)
                sj = idx_v[pl.ds(j * nlane, nlane)]
                dx = plsc.load_gather(px_v, [sj]) - plsc.load_gather(px_v, [ai])
                dy = plsc.load_gather(py_v, [sj]) - plsc.load_gather(py_v, [ai])
                dz = plsc.load_gather(pz_v, [sj]) - plsc.load_gather(pz_v, [ai])
                d2_v[pl.ds(j * nlane, nlane)] = dx * dx + dy * dy + dz * dz
                return c2

            lax.fori_loop(0, chunk // nlane, vec_step, 0)
            pltpu.sync_copy(d2_v, out_hbm.at[pl.ds(off, chunk)])
            return carry

        lax.fori_loop(0, EPW // chunk, chunk_step, 0)

    return dist_k


def _stage0_body(d2_ref, z_ref, emb_ref, r_ref, x0_ref):
    d2 = d2_ref[...]
    ab = d2.shape[0]
    r_ref[...] = jnp.sqrt(d2)

    z = z_ref[...]
    cols = lax.broadcasted_iota(jnp.int32, (ab, ZP), 1)
    oh = (cols == z).astype(jnp.float32)
    x0_ref[...] = jnp.dot(oh, emb_ref[...], preferred_element_type=jnp.float32)


def _pre_body(x_ref, w_ref, x1_ref, xi_ref, y_ref):
    w = w_ref[...]
    x = x_ref[...]
    x1 = _res(x, w[0], w[1, 0], w[2], w[3, 0])
    xi = _res(x1, w[4], w[5, 0], w[6], w[7, 0])
    xi = jnp.dot(_sw(xi), w[8], preferred_element_type=jnp.float32) + w[9, 0]
    yb = _res(x1, w[10], w[11, 0], w[12], w[13, 0])
    yb = jnp.dot(_sw(yb), w[14], preferred_element_type=jnp.float32) + w[15, 0]
    x1_ref[...] = x1
    xi_ref[...] = xi
    y_ref[...] = yb


def _post_body(yj_ref, r_ref, xi_ref, x1_ref, xl_ref, fw_ref, w_ref,
               alpha_ref, lb_ref, out_ref):
    eb = r_ref.shape[1]
    ab = eb // NN
    r = r_ref[...]
    alpha = alpha_ref[0, 0]
    ok = r > 0.0
    xe = jnp.where(ok, -alpha * r, -1.0)
    logt = jnp.log(1.0 - jnp.exp(xe))
    xK = jnp.broadcast_to(xe, (K, eb))
    ltK = jnp.broadcast_to(logt, (K, eb))
    okK = jnp.broadcast_to(ok.astype(jnp.float32), (K, eb))
    n = lax.broadcasted_iota(jnp.int32, (K, 1), 0).astype(jnp.float32)
    lb = lb_ref[...]
    logrbf = lb + n * xK + (K - 1.0 - n) * ltK + xK
    fT = jnp.exp(logrbf) * okK

    wf = lax.dot_general(fT, fw_ref[...], (((0,), (0,)), ((), ())),
                         preferred_element_type=jnp.float32)
    yj = yj_ref[...].astype(jnp.float32)
    xj = jnp.sum((yj * wf).reshape(ab, NN, F), axis=1)

    w = w_ref[...]
    v = xi_ref[...] + xj
    v = _res(v, w[0], w[1, 0], w[2], w[3, 0])
    v = jnp.dot(_sw(v), w[4], preferred_element_type=jnp.float32) + w[5, 0]
    x2 = x1_ref[...] + v
    x2 = _res(x2, w[6], w[7, 0], w[8], w[9, 0])
    out_ref[...] = xl_ref[...] + x2


def _pack_pre(p):
    iw1, ib1, iw2, ib2 = p['in_res'][0]
    rw1, rb1, rw2, rb2 = p['i_res'][0]
    dw, db = p['i_dense']
    jw1, jb1, jw2, jb2 = p['j_res'][0]
    kw, kb = p['j_dense']
    mats = [
        iw1.T, _bias_mat(ib1), iw2.T, _bias_mat(ib2),
        rw1.T, _bias_mat(rb1), rw2.T, _bias_mat(rb2),
        dw.T, _bias_mat(db),
        jw1.T, _bias_mat(jb1), jw2.T, _bias_mat(jb2),
        kw.T, _bias_mat(kb),
    ]
    return jnp.stack(mats)


def _pack_post(p):
    vw1, vb1, vw2, vb2 = p['v_res'][0]
    dw, db = p['v_dense']
    ow1, ob1, ow2, ob2 = p['out_res'][0]
    mats = [
        vw1.T, _bias_mat(vb1), vw2.T, _bias_mat(vb2),
        dw.T, _bias_mat(db),
        ow1.T, _bias_mat(ob1), ow2.T, _bias_mat(ob2),
    ]
    return jnp.stack(mats)


def _bias_mat(b):
    return jnp.zeros((F, F), jnp.float32).at[0, :].set(b)


_AB0 = 1000
_ABP = 1000
_ABC = 200


def _stage0(d2, z2d, emb_p):
    g0 = N // _AB0
    return pl.pallas_call(
        _stage0_body,
        grid=(g0,),
        out_shape=(
            jax.ShapeDtypeStruct((N, NN), jnp.float32),
            jax.ShapeDtypeStruct((N, F), jnp.float32),
        ),
        in_specs=[
            pl.BlockSpec((_AB0, NN), lambda i: (i, 0)),
            pl.BlockSpec((_AB0, 1), lambda i: (i, 0)),
            pl.BlockSpec((ZP, F), lambda i: (0, 0)),
        ],
        out_specs=(
            pl.BlockSpec((_AB0, NN), lambda i: (i, 0)),
            pl.BlockSpec((_AB0, F), lambda i: (i, 0)),
        ),
    )(d2, z2d, emb_p)


def _pre(x, wpack):
    g = N // _ABP
    return pl.pallas_call(
        _pre_body,
        grid=(g,),
        out_shape=(
            jax.ShapeDtypeStruct((N, F), jnp.float32),
            jax.ShapeDtypeStruct((N, F), jnp.float32),
            jax.ShapeDtypeStruct((N, F), jnp.float32),
        ),
        in_specs=[
            pl.BlockSpec((_ABP, F), lambda i: (i, 0)),
            pl.BlockSpec((16, F, F), lambda i: (0, 0, 0)),
        ],
        out_specs=(
            pl.BlockSpec((_ABP, F), lambda i: (i, 0)),
            pl.BlockSpec((_ABP, F), lambda i: (i, 0)),
            pl.BlockSpec((_ABP, F), lambda i: (i, 0)),
        ),
    )(x, wpack)


def _post(yj, r1, xi, x1, xl, fwT, wpack, alpha2d, lbK):
    na = xi.shape[0]
    g = na // _ABC
    return pl.pallas_call(
        _post_body,
        grid=(g,),
        out_shape=jax.ShapeDtypeStruct((na, F), jnp.float32),
        in_specs=[
            pl.BlockSpec((_ABC * NN, F), lambda i: (i, 0)),
            pl.BlockSpec((1, _ABC * NN), lambda i: (0, i)),
            pl.BlockSpec((_ABC, F), lambda i: (i, 0)),
            pl.BlockSpec((_ABC, F), lambda i: (i, 0)),
            pl.BlockSpec((_ABC, F), lambda i: (i, 0)),
            pl.BlockSpec((K, F), lambda i: (0, 0)),
            pl.BlockSpec((10, F, F), lambda i: (0, 0, 0)),
            pl.BlockSpec((1, 1), lambda i: (0, 0)),
            pl.BlockSpec((K, 1), lambda i: (0, 0)),
        ],
        out_specs=pl.BlockSpec((_ABC, F), lambda i: (i, 0)),
    )(yj, r1, xi, x1, xl, fwT, wpack, alpha2d, lbK)


def _gather_rows(idx, table, d, chunk):
    return _make_sc_gather(d, chunk, table.dtype.name, idx.shape[0])(idx, table)


def kernel(atomic_numbers, positions, cell, cell_offset, neighbors,
           neighbor_mask, atom_mask, emb, alpha, params):
    idx = neighbors[0].reshape(E).astype(jnp.int32)
    z2d = atomic_numbers[0].reshape(N, 1).astype(jnp.int32)
    emb_p = jnp.pad(emb, ((0, ZP - MAXZ), (0, 0)))
    alpha2d = alpha.reshape(1, 1)
    lbK = jnp.asarray(_LOGBINOM).reshape(K, 1)

    px, py, pz = (positions[0, :, i] for i in range(3))
    d2 = _make_sc_dist(2000)(idx, px, py, pz).reshape(N, NN)
    r, x = _stage0(d2, z2d, emb_p)
    r1 = r.reshape(1, E)

    half, nh = E // 2, N // 2
    for p in params:
        x1, xi, y = _pre(x, _pack_pre(p))
        fwT, wpost = p['filt_W'].T, _pack_post(p)
        xs = []
        for h in range(2):
            se = slice(h * half, (h + 1) * half)
            sa = slice(h * nh, (h + 1) * nh)
            yj = _gather_rows(idx[se], y, F, 200)
            xs.append(_post(yj, r1[:, se], xi[sa], x1[sa], x[sa],
                            fwT, wpost, alpha2d, lbK))
        x = jnp.concatenate(xs, axis=0)

    return (x[None], r[None])


# speedup vs baseline: 1.1148x; 1.1148x over previous
import functools
import math

import jax
import jax.numpy as jnp
import numpy as np
from jax import lax
from jax.experimental import pallas as pl
from jax.experimental.pallas import tpu as pltpu
from jax.experimental.pallas import tpu_sc as plsc

N, NN, F, K, MAXZ = 10000, 32, 128, 32, 87
E = N * NN
ZP = 96
PD = 128

NC, NS = 2, 16
NW = NC * NS
EPW = E // NW

_LOGBINOM = np.array(
    [math.lgamma(K) - math.lgamma(i + 1.0) - math.lgamma(K - i) for i in range(K)],
    dtype=np.float32,
)


def _sw(x):
    return x * jax.nn.sigmoid(x)


def _res(x, w1, b1, w2, b2):
    y = jnp.dot(_sw(x), w1, preferred_element_type=jnp.float32) + b1
    y = jnp.dot(_sw(y), w2, preferred_element_type=jnp.float32) + b2
    return x + y


@functools.lru_cache(maxsize=None)
def _make_sc_gather(d, chunk, dtype_name, ne):
    dtype = jnp.dtype(dtype_name)
    mesh = plsc.VectorSubcoreMesh(core_axis_name="c", subcore_axis_name="s")
    epw = ne // NW
    nch = epw // chunk
    npairs = nch // 2
    odd = nch % 2 == 1
    assert epw % chunk == 0 and chunk % 8 == 0 and nch >= 2

    @functools.partial(
        pl.kernel,
        mesh=mesh,
        out_type=jax.ShapeDtypeStruct((ne, d), dtype),
        scratch_types=[
            pltpu.VMEM((epw,), jnp.int32),
            pltpu.VMEM((2, chunk, d), dtype),
            pltpu.SemaphoreType.DMA,
            pltpu.SemaphoreType.DMA,
            pltpu.SemaphoreType.DMA,
            pltpu.SemaphoreType.DMA,
        ],
    )
    def gather_k(idx_hbm, table_hbm, out_hbm, idx_v, rows_v, g0, g1, w0, w1):
        wid = lax.axis_index("s") * NC + lax.axis_index("c")
        base = wid * epw
        pltpu.sync_copy(idx_hbm.at[pl.ds(base, epw)], idx_v)

        def fire_g(c, slot, sem):
            pltpu.async_copy(
                table_hbm.at[idx_v.at[pl.ds(c * chunk, chunk)]],
                rows_v.at[slot], sem)

        def wait_g(slot, sem):
            pltpu.make_async_copy(
                table_hbm.at[idx_v.at[pl.ds(0, chunk)]], rows_v.at[slot], sem
            ).wait()

        def fire_w(c, slot, sem):
            pltpu.async_copy(
                rows_v.at[slot], out_hbm.at[pl.ds(base + c * chunk, chunk)],
                sem)

        def wait_w(slot, sem):
            pltpu.make_async_copy(
                rows_v.at[slot], out_hbm.at[pl.ds(base, chunk)], sem
            ).wait()

        fire_g(0, 0, g0)

        def pair(g, carry):
            c0 = 2 * g
            wait_g(0, g0)
            fire_w(c0, 0, w0)

            @pl.when(g > 0)
            def _():
                wait_w(1, w1)

            fire_g(c0 + 1, 1, g1)
            wait_g(1, g1)
            fire_w(c0 + 1, 1, w1)
            wait_w(0, w0)

            @pl.when(c0 + 2 < nch)
            def _():
                fire_g(c0 + 2, 0, g0)

            return carry

        lax.fori_loop(0, npairs, pair, 0)
        if odd:
            wait_g(0, g0)
            fire_w(nch - 1, 0, w0)
            wait_w(0, w0)
        if npairs > 0:
            wait_w(1, w1)

    return gather_k


def _make_sc_dist(chunk):
    mesh = plsc.VectorSubcoreMesh(core_axis_name="c", subcore_axis_name="s")
    nlane = 16

    @functools.partial(
        pl.kernel,
        mesh=mesh,
        out_type=jax.ShapeDtypeStruct((E,), jnp.float32),
        compiler_params=pltpu.CompilerParams(needs_layout_passes=False),
        scratch_types=[
            pltpu.VMEM((N,), jnp.float32),
            pltpu.VMEM((N,), jnp.float32),
            pltpu.VMEM((N,), jnp.float32),
            pltpu.VMEM((chunk,), jnp.int32),
            pltpu.VMEM((chunk,), jnp.float32),
        ],
    )
    def dist_k(idx_hbm, px_hbm, py_hbm, pz_hbm, out_hbm,
               px_v, py_v, pz_v, idx_v, d2_v):
        wid = lax.axis_index("s") * NC + lax.axis_index("c")
        base = wid * EPW
        pltpu.sync_copy(px_hbm, px_v)
        pltpu.sync_copy(py_hbm, py_v)
        pltpu.sync_copy(pz_hbm, pz_v)

        def chunk_step(i, carry):
            off = base + i * chunk
            pltpu.sync_copy(idx_hbm.at[pl.ds(off, chunk)], idx_v)

            def vec_step(j, c2):
                jv = lax.iota(jnp.int32, nlane) + (off + j * nlane)
                ai = lax.shift_right_logical(jv, 5)
                sj = idx_v[pl.ds(j * nlane, nlane)]
                dx = plsc.load_gather(px_v, [sj]) - plsc.load_gather(px_v, [ai])
                dy = plsc.load_gather(py_v, [sj]) - plsc.load_gather(py_v, [ai])
                dz = plsc.load_gather(pz_v, [sj]) - plsc.load_gather(pz_v, [ai])
                d2_v[pl.ds(j * nlane, nlane)] = dx * dx + dy * dy + dz * dz
                return c2

            lax.fori_loop(0, chunk // nlane, vec_step, 0)
            pltpu.sync_copy(d2_v, out_hbm.at[pl.ds(off, chunk)])
            return carry

        lax.fori_loop(0, EPW // chunk, chunk_step, 0)

    return dist_k


def _stage0_body(d2_ref, z_ref, emb_ref, r_ref, x0_ref):
    d2 = d2_ref[...]
    ab = d2.shape[0]
    r_ref[...] = jnp.sqrt(d2)

    z = z_ref[...]
    cols = lax.broadcasted_iota(jnp.int32, (ab, ZP), 1)
    oh = (cols == z).astype(jnp.float32)
    x0_ref[...] = jnp.dot(oh, emb_ref[...], preferred_element_type=jnp.float32)


def _pre_body(x_ref, w_ref, x1_ref, xi_ref, y_ref):
    w = w_ref[...]
    x = x_ref[...]
    x1 = _res(x, w[0], w[1, 0], w[2], w[3, 0])
    xi = _res(x1, w[4], w[5, 0], w[6], w[7, 0])
    xi = jnp.dot(_sw(xi), w[8], preferred_element_type=jnp.float32) + w[9, 0]
    yb = _res(x1, w[10], w[11, 0], w[12], w[13, 0])
    yb = jnp.dot(_sw(yb), w[14], preferred_element_type=jnp.float32) + w[15, 0]
    x1_ref[...] = x1
    xi_ref[...] = xi
    y_ref[...] = yb


def _post_body(yj_ref, r_ref, xi_ref, x1_ref, xl_ref, fw_ref, w_ref,
               alpha_ref, lb_ref, out_ref):
    eb = r_ref.shape[1]
    ab = eb // NN
    r = r_ref[...]
    alpha = alpha_ref[0, 0]
    ok = r > 0.0
    xe = jnp.where(ok, -alpha * r, -1.0)
    logt = jnp.log(1.0 - jnp.exp(xe))
    xK = jnp.broadcast_to(xe, (K, eb))
    ltK = jnp.broadcast_to(logt, (K, eb))
    okK = jnp.broadcast_to(ok.astype(jnp.float32), (K, eb))
    n = lax.broadcasted_iota(jnp.int32, (K, 1), 0).astype(jnp.float32)
    lb = lb_ref[...]
    logrbf = lb + n * xK + (K - 1.0 - n) * ltK + xK
    fT = jnp.exp(logrbf) * okK

    wf = lax.dot_general(fT, fw_ref[...], (((0,), (0,)), ((), ())),
                         preferred_element_type=jnp.float32)
    yj = yj_ref[...].astype(jnp.float32)
    xj = jnp.sum((yj * wf).reshape(ab, NN, F), axis=1)

    w = w_ref[...]
    v = xi_ref[...] + xj
    v = _res(v, w[0], w[1, 0], w[2], w[3, 0])
    v = jnp.dot(_sw(v), w[4], preferred_element_type=jnp.float32) + w[5, 0]
    x2 = x1_ref[...] + v
    x2 = _res(x2, w[6], w[7, 0], w[8], w[9, 0])
    out_ref[...] = xl_ref[...] + x2


def _pack_pre(p):
    iw1, ib1, iw2, ib2 = p['in_res'][0]
    rw1, rb1, rw2, rb2 = p['i_res'][0]
    dw, db = p['i_dense']
    jw1, jb1, jw2, jb2 = p['j_res'][0]
    kw, kb = p['j_dense']
    mats = [
        iw1.T, _bias_mat(ib1), iw2.T, _bias_mat(ib2),
        rw1.T, _bias_mat(rb1), rw2.T, _bias_mat(rb2),
        dw.T, _bias_mat(db),
        jw1.T, _bias_mat(jb1), jw2.T, _bias_mat(jb2),
        kw.T, _bias_mat(kb),
    ]
    return jnp.stack(mats)


def _pack_post(p):
    vw1, vb1, vw2, vb2 = p['v_res'][0]
    dw, db = p['v_dense']
    ow1, ob1, ow2, ob2 = p['out_res'][0]
    mats = [
        vw1.T, _bias_mat(vb1), vw2.T, _bias_mat(vb2),
        dw.T, _bias_mat(db),
        ow1.T, _bias_mat(ob1), ow2.T, _bias_mat(ob2),
    ]
    return jnp.stack(mats)


def _bias_mat(b):
    return jnp.zeros((F, F), jnp.float32).at[0, :].set(b)


_AB0 = 1000
_ABP = 1000
_ABC = 400


def _stage0(d2, z2d, emb_p):
    g0 = N // _AB0
    return pl.pallas_call(
        _stage0_body,
        grid=(g0,),
        out_shape=(
            jax.ShapeDtypeStruct((N, NN), jnp.float32),
            jax.ShapeDtypeStruct((N, F), jnp.float32),
        ),
        in_specs=[
            pl.BlockSpec((_AB0, NN), lambda i: (i, 0)),
            pl.BlockSpec((_AB0, 1), lambda i: (i, 0)),
            pl.BlockSpec((ZP, F), lambda i: (0, 0)),
        ],
        out_specs=(
            pl.BlockSpec((_AB0, NN), lambda i: (i, 0)),
            pl.BlockSpec((_AB0, F), lambda i: (i, 0)),
        ),
    )(d2, z2d, emb_p)


def _pre(x, wpack):
    g = N // _ABP
    return pl.pallas_call(
        _pre_body,
        grid=(g,),
        out_shape=(
            jax.ShapeDtypeStruct((N, F), jnp.float32),
            jax.ShapeDtypeStruct((N, F), jnp.float32),
            jax.ShapeDtypeStruct((N, F), jnp.float32),
        ),
        in_specs=[
            pl.BlockSpec((_ABP, F), lambda i: (i, 0)),
            pl.BlockSpec((16, F, F), lambda i: (0, 0, 0)),
        ],
        out_specs=(
            pl.BlockSpec((_ABP, F), lambda i: (i, 0)),
            pl.BlockSpec((_ABP, F), lambda i: (i, 0)),
            pl.BlockSpec((_ABP, F), lambda i: (i, 0)),
        ),
    )(x, wpack)


def _post(yj, r1, xi, x1, xl, fwT, wpack, alpha2d, lbK):
    na = xi.shape[0]
    g = na // _ABC
    return pl.pallas_call(
        _post_body,
        grid=(g,),
        out_shape=jax.ShapeDtypeStruct((na, F), jnp.float32),
        in_specs=[
            pl.BlockSpec((_ABC * NN, F), lambda i: (i, 0)),
            pl.BlockSpec((1, _ABC * NN), lambda i: (0, i)),
            pl.BlockSpec((_ABC, F), lambda i: (i, 0)),
            pl.BlockSpec((_ABC, F), lambda i: (i, 0)),
            pl.BlockSpec((_ABC, F), lambda i: (i, 0)),
            pl.BlockSpec((K, F), lambda i: (0, 0)),
            pl.BlockSpec((10, F, F), lambda i: (0, 0, 0)),
            pl.BlockSpec((1, 1), lambda i: (0, 0)),
            pl.BlockSpec((K, 1), lambda i: (0, 0)),
        ],
        out_specs=pl.BlockSpec((_ABC, F), lambda i: (i, 0)),
    )(yj, r1, xi, x1, xl, fwT, wpack, alpha2d, lbK)


def _gather_rows(idx, table, d, chunk):
    return _make_sc_gather(d, chunk, table.dtype.name, idx.shape[0])(idx, table)


def kernel(atomic_numbers, positions, cell, cell_offset, neighbors,
           neighbor_mask, atom_mask, emb, alpha, params):
    idx = neighbors[0].reshape(E).astype(jnp.int32)
    z2d = atomic_numbers[0].reshape(N, 1).astype(jnp.int32)
    emb_p = jnp.pad(emb, ((0, ZP - MAXZ), (0, 0)))
    alpha2d = alpha.reshape(1, 1)
    lbK = jnp.asarray(_LOGBINOM).reshape(K, 1)

    px, py, pz = (positions[0, :, i] for i in range(3))
    d2 = _make_sc_dist(2000)(idx, px, py, pz).reshape(N, NN)
    r, x = _stage0(d2, z2d, emb_p)
    r1 = r.reshape(1, E)

    for p in params:
        x1, xi, y = _pre(x, _pack_pre(p))
        yj = _gather_rows(idx, y, F, 400)
        x = _post(yj, r1, xi, x1, x, p['filt_W'].T, _pack_post(p),
                  alpha2d, lbK)

    return (x[None], r[None])


# speedup vs baseline: 1.1214x; 1.0058x over previous
import functools
import math

import jax
import jax.numpy as jnp
import numpy as np
from jax import lax
from jax.experimental import pallas as pl
from jax.experimental.pallas import tpu as pltpu
from jax.experimental.pallas import tpu_sc as plsc

N, NN, F, K, MAXZ = 10000, 32, 128, 32, 87
E = N * NN
ZP = 96
PD = 128

NC, NS = 2, 16
NW = NC * NS
EPW = E // NW

_LOGBINOM = np.array(
    [math.lgamma(K) - math.lgamma(i + 1.0) - math.lgamma(K - i) for i in range(K)],
    dtype=np.float32,
)


def _sw(x):
    return x * jax.nn.sigmoid(x)


def _res(x, w1, b1, w2, b2):
    y = jnp.dot(_sw(x), w1, preferred_element_type=jnp.float32) + b1
    y = jnp.dot(_sw(y), w2, preferred_element_type=jnp.float32) + b2
    return x + y


@functools.lru_cache(maxsize=None)
def _make_sc_gather(d, chunk, dtype_name, ne):
    dtype = jnp.dtype(dtype_name)
    mesh = plsc.VectorSubcoreMesh(core_axis_name="c", subcore_axis_name="s")
    epw = ne // NW
    nch = epw // chunk
    npairs = nch // 2
    odd = nch % 2 == 1
    assert epw % chunk == 0 and chunk % 8 == 0 and nch >= 2

    @functools.partial(
        pl.kernel,
        mesh=mesh,
        out_type=jax.ShapeDtypeStruct((ne, d), dtype),
        scratch_types=[
            pltpu.VMEM((epw,), jnp.int32),
            pltpu.VMEM((2, chunk, d), dtype),
            pltpu.SemaphoreType.DMA,
            pltpu.SemaphoreType.DMA,
            pltpu.SemaphoreType.DMA,
            pltpu.SemaphoreType.DMA,
        ],
    )
    def gather_k(idx_hbm, table_hbm, out_hbm, idx_v, rows_v, g0, g1, w0, w1):
        wid = lax.axis_index("s") * NC + lax.axis_index("c")
        base = wid * epw
        pltpu.sync_copy(idx_hbm.at[pl.ds(base, epw)], idx_v)

        def fire_g(c, slot, sem):
            pltpu.async_copy(
                table_hbm.at[idx_v.at[pl.ds(c * chunk, chunk)]],
                rows_v.at[slot], sem)

        def wait_g(slot, sem):
            pltpu.make_async_copy(
                table_hbm.at[idx_v.at[pl.ds(0, chunk)]], rows_v.at[slot], sem
            ).wait()

        def fire_w(c, slot, sem):
            pltpu.async_copy(
                rows_v.at[slot], out_hbm.at[pl.ds(base + c * chunk, chunk)],
                sem)

        def wait_w(slot, sem):
            pltpu.make_async_copy(
                rows_v.at[slot], out_hbm.at[pl.ds(base, chunk)], sem
            ).wait()

        fire_g(0, 0, g0)

        def pair(g, carry):
            c0 = 2 * g
            wait_g(0, g0)
            fire_w(c0, 0, w0)

            @pl.when(g > 0)
            def _():
                wait_w(1, w1)

            fire_g(c0 + 1, 1, g1)
            wait_g(1, g1)
            fire_w(c0 + 1, 1, w1)
            wait_w(0, w0)

            @pl.when(c0 + 2 < nch)
            def _():
                fire_g(c0 + 2, 0, g0)

            return carry

        lax.fori_loop(0, npairs, pair, 0)
        if odd:
            wait_g(0, g0)
            fire_w(nch - 1, 0, w0)
            wait_w(0, w0)
        if npairs > 0:
            wait_w(1, w1)

    return gather_k


def _make_sc_dist(chunk):
    mesh = plsc.VectorSubcoreMesh(core_axis_name="c", subcore_axis_name="s")
    nlane = 16

    @functools.partial(
        pl.kernel,
        mesh=mesh,
        out_type=jax.ShapeDtypeStruct((E,), jnp.float32),
        compiler_params=pltpu.CompilerParams(needs_layout_passes=False),
        scratch_types=[
            pltpu.VMEM((N,), jnp.float32),
            pltpu.VMEM((N,), jnp.float32),
            pltpu.VMEM((N,), jnp.float32),
            pltpu.VMEM((chunk,), jnp.int32),
            pltpu.VMEM((chunk,), jnp.float32),
        ],
    )
    def dist_k(idx_hbm, px_hbm, py_hbm, pz_hbm, out_hbm,
               px_v, py_v, pz_v, idx_v, d2_v):
        wid = lax.axis_index("s") * NC + lax.axis_index("c")
        base = wid * EPW
        pltpu.sync_copy(px_hbm, px_v)
        pltpu.sync_copy(py_hbm, py_v)
        pltpu.sync_copy(pz_hbm, pz_v)

        def chunk_step(i, carry):
            off = base + i * chunk
            pltpu.sync_copy(idx_hbm.at[pl.ds(off, chunk)], idx_v)

            def vec_step(j, c2):
                jv = lax.iota(jnp.int32, nlane) + (off + j * nlane)
                ai = lax.shift_right_logical(jv, 5)
                sj = idx_v[pl.ds(j * nlane, nlane)]
                dx = plsc.load_gather(px_v, [sj]) - plsc.load_gather(px_v, [ai])
                dy = plsc.load_gather(py_v, [sj]) - plsc.load_gather(py_v, [ai])
                dz = plsc.load_gather(pz_v, [sj]) - plsc.load_gather(pz_v, [ai])
                d2_v[pl.ds(j * nlane, nlane)] = dx * dx + dy * dy + dz * dz
                return c2

            lax.fori_loop(0, chunk // nlane, vec_step, 0)
            pltpu.sync_copy(d2_v, out_hbm.at[pl.ds(off, chunk)])
            return carry

        lax.fori_loop(0, EPW // chunk, chunk_step, 0)

    return dist_k


def _pre_mats(x, w):
    x1 = _res(x, w[0], w[1, 0], w[2], w[3, 0])
    xi = _res(x1, w[4], w[5, 0], w[6], w[7, 0])
    xi = jnp.dot(_sw(xi), w[8], preferred_element_type=jnp.float32) + w[9, 0]
    yb = _res(x1, w[10], w[11, 0], w[12], w[13, 0])
    yb = jnp.dot(_sw(yb), w[14], preferred_element_type=jnp.float32) + w[15, 0]
    return x1, xi, yb


def _post_core(yj_ref, r_ref, xi, x1, xl, fw_ref, w, alpha_ref, lb_ref):
    eb = r_ref.shape[1]
    ab = eb // NN
    r = r_ref[...]
    alpha = alpha_ref[0, 0]
    ok = r > 0.0
    xe = jnp.where(ok, -alpha * r, -1.0)
    logt = jnp.log(1.0 - jnp.exp(xe))
    xK = jnp.broadcast_to(xe, (K, eb))
    ltK = jnp.broadcast_to(logt, (K, eb))
    okK = jnp.broadcast_to(ok.astype(jnp.float32), (K, eb))
    n = lax.broadcasted_iota(jnp.int32, (K, 1), 0).astype(jnp.float32)
    lb = lb_ref[...]
    logrbf = lb + n * xK + (K - 1.0 - n) * ltK + xK
    fT = jnp.exp(logrbf) * okK

    wf = lax.dot_general(fT, fw_ref[...], (((0,), (0,)), ((), ())),
                         preferred_element_type=jnp.float32)
    yj = yj_ref[...].astype(jnp.float32)
    xj = jnp.sum((yj * wf).reshape(ab, NN, F), axis=1)

    v = xi + xj
    v = _res(v, w[0], w[1, 0], w[2], w[3, 0])
    v = jnp.dot(_sw(v), w[4], preferred_element_type=jnp.float32) + w[5, 0]
    x2 = x1 + v
    x2 = _res(x2, w[6], w[7, 0], w[8], w[9, 0])
    return xl + x2


def _first_body(d2_ref, z_ref, emb_ref, w_ref, r_ref, x0_ref,
                x1_ref, xi_ref, y_ref):
    d2 = d2_ref[...]
    ab = d2.shape[0]
    r_ref[...] = jnp.sqrt(d2)

    z = z_ref[...]
    cols = lax.broadcasted_iota(jnp.int32, (ab, ZP), 1)
    oh = (cols == z).astype(jnp.float32)
    x0 = jnp.dot(oh, emb_ref[...], preferred_element_type=jnp.float32)
    x0_ref[...] = x0
    x1, xi, yb = _pre_mats(x0, w_ref[...])
    x1_ref[...] = x1
    xi_ref[...] = xi
    y_ref[...] = yb


def _mid_body(yj_ref, r_ref, xi_ref, x1_ref, xl_ref, fw_ref, wpost_ref,
              wpre_ref, alpha_ref, lb_ref,
              xout_ref, x1b_ref, xib_ref, yb_ref):
    xnew = _post_core(yj_ref, r_ref, xi_ref[...], x1_ref[...], xl_ref[...],
                      fw_ref, wpost_ref[...], alpha_ref, lb_ref)
    xout_ref[...] = xnew
    x1b, xib, yb = _pre_mats(xnew, wpre_ref[...])
    x1b_ref[...] = x1b
    xib_ref[...] = xib
    yb_ref[...] = yb


def _post_body(yj_ref, r_ref, xi_ref, x1_ref, xl_ref, fw_ref, w_ref,
               alpha_ref, lb_ref, out_ref):
    out_ref[...] = _post_core(yj_ref, r_ref, xi_ref[...], x1_ref[...],
                              xl_ref[...], fw_ref, w_ref[...],
                              alpha_ref, lb_ref)


def _pack_pre(p):
    iw1, ib1, iw2, ib2 = p['in_res'][0]
    rw1, rb1, rw2, rb2 = p['i_res'][0]
    dw, db = p['i_dense']
    jw1, jb1, jw2, jb2 = p['j_res'][0]
    kw, kb = p['j_dense']
    mats = [
        iw1.T, _bias_mat(ib1), iw2.T, _bias_mat(ib2),
        rw1.T, _bias_mat(rb1), rw2.T, _bias_mat(rb2),
        dw.T, _bias_mat(db),
        jw1.T, _bias_mat(jb1), jw2.T, _bias_mat(jb2),
        kw.T, _bias_mat(kb),
    ]
    return jnp.stack(mats)


def _pack_post(p):
    vw1, vb1, vw2, vb2 = p['v_res'][0]
    dw, db = p['v_dense']
    ow1, ob1, ow2, ob2 = p['out_res'][0]
    mats = [
        vw1.T, _bias_mat(vb1), vw2.T, _bias_mat(vb2),
        dw.T, _bias_mat(db),
        ow1.T, _bias_mat(ob1), ow2.T, _bias_mat(ob2),
    ]
    return jnp.stack(mats)


def _bias_mat(b):
    return jnp.zeros((F, F), jnp.float32).at[0, :].set(b)


_AB0 = 1000
_ABP = 1000
_ABC = 400


def _first(d2, z2d, emb_p, wpre):
    g0 = N // _AB0
    fs = pl.BlockSpec((_AB0, F), lambda i: (i, 0))
    return pl.pallas_call(
        _first_body,
        grid=(g0,),
        out_shape=(
            jax.ShapeDtypeStruct((N, NN), jnp.float32),
            jax.ShapeDtypeStruct((N, F), jnp.float32),
            jax.ShapeDtypeStruct((N, F), jnp.float32),
            jax.ShapeDtypeStruct((N, F), jnp.float32),
            jax.ShapeDtypeStruct((N, F), jnp.float32),
        ),
        in_specs=[
            pl.BlockSpec((_AB0, NN), lambda i: (i, 0)),
            pl.BlockSpec((_AB0, 1), lambda i: (i, 0)),
            pl.BlockSpec((ZP, F), lambda i: (0, 0)),
            pl.BlockSpec((16, F, F), lambda i: (0, 0, 0)),
        ],
        out_specs=(
            pl.BlockSpec((_AB0, NN), lambda i: (i, 0)), fs, fs, fs, fs,
        ),
    )(d2, z2d, emb_p, wpre)


def _mid(yj, r1, xi, x1, xl, fwT, wpost, wpre, alpha2d, lbK):
    g = N // _ABC
    fs = pl.BlockSpec((_ABC, F), lambda i: (i, 0))
    return pl.pallas_call(
        _mid_body,
        grid=(g,),
        out_shape=(
            jax.ShapeDtypeStruct((N, F), jnp.float32),
            jax.ShapeDtypeStruct((N, F), jnp.float32),
            jax.ShapeDtypeStruct((N, F), jnp.float32),
            jax.ShapeDtypeStruct((N, F), jnp.float32),
        ),
        in_specs=[
            pl.BlockSpec((_ABC * NN, F), lambda i: (i, 0)),
            pl.BlockSpec((1, _ABC * NN), lambda i: (0, i)),
            fs, fs, fs,
            pl.BlockSpec((K, F), lambda i: (0, 0)),
            pl.BlockSpec((10, F, F), lambda i: (0, 0, 0)),
            pl.BlockSpec((16, F, F), lambda i: (0, 0, 0)),
            pl.BlockSpec((1, 1), lambda i: (0, 0)),
            pl.BlockSpec((K, 1), lambda i: (0, 0)),
        ],
        out_specs=(fs, fs, fs, fs),
    )(yj, r1, xi, x1, xl, fwT, wpost, wpre, alpha2d, lbK)


def _post(yj, r1, xi, x1, xl, fwT, wpack, alpha2d, lbK):
    na = xi.shape[0]
    g = na // _ABC
    return pl.pallas_call(
        _post_body,
        grid=(g,),
        out_shape=jax.ShapeDtypeStruct((na, F), jnp.float32),
        in_specs=[
            pl.BlockSpec((_ABC * NN, F), lambda i: (i, 0)),
            pl.BlockSpec((1, _ABC * NN), lambda i: (0, i)),
            pl.BlockSpec((_ABC, F), lambda i: (i, 0)),
            pl.BlockSpec((_ABC, F), lambda i: (i, 0)),
            pl.BlockSpec((_ABC, F), lambda i: (i, 0)),
            pl.BlockSpec((K, F), lambda i: (0, 0)),
            pl.BlockSpec((10, F, F), lambda i: (0, 0, 0)),
            pl.BlockSpec((1, 1), lambda i: (0, 0)),
            pl.BlockSpec((K, 1), lambda i: (0, 0)),
        ],
        out_specs=pl.BlockSpec((_ABC, F), lambda i: (i, 0)),
    )(yj, r1, xi, x1, xl, fwT, wpack, alpha2d, lbK)


def _gather_rows(idx, table, d, chunk):
    return _make_sc_gather(d, chunk, table.dtype.name, idx.shape[0])(idx, table)


def kernel(atomic_numbers, positions, cell, cell_offset, neighbors,
           neighbor_mask, atom_mask, emb, alpha, params):
    idx = neighbors[0].reshape(E).astype(jnp.int32)
    z2d = atomic_numbers[0].reshape(N, 1).astype(jnp.int32)
    emb_p = jnp.pad(emb, ((0, ZP - MAXZ), (0, 0)))
    alpha2d = alpha.reshape(1, 1)
    lbK = jnp.asarray(_LOGBINOM).reshape(K, 1)

    px, py, pz = (positions[0, :, i] for i in range(3))
    d2 = _make_sc_dist(2000)(idx, px, py, pz).reshape(N, NN)
    p1, p2 = params
    r, x0, x1, xi, y = _first(d2, z2d, emb_p, _pack_pre(p1))
    r1 = r.reshape(1, E)

    yj = _gather_rows(idx, y, F, 400)
    x, x1b, xib, yb = _mid(yj, r1, xi, x1, x0, p1['filt_W'].T,
                           _pack_post(p1), _pack_pre(p2), alpha2d, lbK)
    yj2 = _gather_rows(idx, yb, F, 400)
    xfin = _post(yj2, r1, xib, x1b, x, p2['filt_W'].T, _pack_post(p2),
                 alpha2d, lbK)

    return (xfin[None], r[None])
